# baseline (speedup 1.0000x reference)
import functools

import jax
import jax.numpy as jnp
from jax import lax
from jax.experimental import pallas as pl
from jax.experimental.pallas import tpu as pltpu
from jax.experimental.pallas import tpu_sc as plsc

B = 16384
D = 32
NW = 32
BPW = B // NW
CHUNK = 128


def _gather_body(ut_hbm, it_hbm, iu_hbm, ii_hbm, u_out, i_out,
                 iu_v, ii_v, u_rows, i_rows, sem):
    wid = lax.axis_index("s") * 2 + lax.axis_index("c")
    base = wid * BPW
    pltpu.sync_copy(iu_hbm.at[pl.ds(base, BPW)], iu_v)
    pltpu.sync_copy(ii_hbm.at[pl.ds(base, BPW)], ii_v)
    copies = []
    for j in range(BPW // CHUNK):
        s = pl.ds(j * CHUNK, CHUNK)
        copies.append(pltpu.async_copy(ut_hbm.at[iu_v.at[s]], u_rows.at[s], sem))
        copies.append(pltpu.async_copy(it_hbm.at[ii_v.at[s]], i_rows.at[s], sem))
    for c in copies:
        c.wait()
    pltpu.sync_copy(u_rows, u_out.at[pl.ds(base, BPW)])
    pltpu.sync_copy(i_rows, i_out.at[pl.ds(base, BPW)])


@jax.jit
def _sc_gather(user_table, item_table, idx_u, idx_i):
    mesh = plsc.VectorSubcoreMesh(core_axis_name="c", subcore_axis_name="s")
    emb = jax.ShapeDtypeStruct((B, D), jnp.float32)
    return pl.kernel(
        _gather_body,
        mesh=mesh,
        compiler_params=pltpu.CompilerParams(use_tc_tiling_on_sc=False),
        out_type=(emb, emb),
        scratch_types=[
            pltpu.VMEM((BPW,), jnp.int32),
            pltpu.VMEM((BPW,), jnp.int32),
            pltpu.VMEM((BPW, D), jnp.float32),
            pltpu.VMEM((BPW, D), jnp.float32),
            pltpu.SemaphoreType.DMA,
        ],
    )(user_table, item_table, idx_u, idx_i)


TB = 4096


def _mlp_body(u_ref, i_ref, w1u_ref, w1i_ref, b1_ref, w2_ref, b2_ref,
              w3_ref, b3_ref, o_ref):
    h = jnp.dot(u_ref[...], w1u_ref[...], preferred_element_type=jnp.float32)
    h = h + jnp.dot(i_ref[...], w1i_ref[...], preferred_element_type=jnp.float32)
    h = jax.nn.relu(h + b1_ref[...])
    h = jax.nn.relu(
        jnp.dot(h, w2_ref[...], preferred_element_type=jnp.float32) + b2_ref[...])
    o_ref[...] = (
        jnp.dot(h, w3_ref[...], preferred_element_type=jnp.float32) + b3_ref[...])


@jax.jit
def _tc_mlp(u_emb, i_emb, W1u, W1i, b1, W2, b2, W3, b3):
    full = lambda r, c: pl.BlockSpec((r, c), lambda i: (0, 0))
    return pl.pallas_call(
        _mlp_body,
        grid=(B // TB,),
        in_specs=[
            pl.BlockSpec((TB, D), lambda i: (i, 0)),
            pl.BlockSpec((TB, D), lambda i: (i, 0)),
            full(D, 64), full(D, 64), full(1, 64),
            full(64, 16), full(1, 16),
            full(16, 1), full(1, 1),
        ],
        out_specs=pl.BlockSpec((TB, 1), lambda i: (i, 0)),
        out_shape=jax.ShapeDtypeStruct((B, 1), jnp.float32),
    )(u_emb, i_emb, W1u, W1i, b1, W2, b2, W3, b3)


def kernel(inputs, user_table, item_table, W1, b1, W2, b2, W3, b3):
    idx_u = inputs[:, 0]
    idx_i = inputs[:, 1]
    u_emb, i_emb = _sc_gather(user_table, item_table, idx_u, idx_i)
    return _tc_mlp(
        u_emb, i_emb,
        W1[:D, :], W1[D:, :], b1.reshape(1, 64),
        W2, b2.reshape(1, 16),
        W3, b3.reshape(1, 1),
    )
